# baseline (speedup 1.0000x reference)
import functools
import math

import jax
import jax.numpy as jnp
from jax import lax
from jax.experimental import pallas as pl
from jax.experimental.pallas import tpu as pltpu
from jax.experimental.pallas import tpu_sc as plsc

D = 64
NW = 32
CHUNK = 512
LANES = 16
SCALE = math.sqrt(D)


def _build(n_idx):
    per_w = n_idx // NW
    nch = per_w // CHUNK
    mesh = plsc.VectorSubcoreMesh(core_axis_name="c", subcore_axis_name="s")

    @functools.partial(
        pl.kernel,
        out_type=jax.ShapeDtypeStruct((n_idx, D), jnp.float32),
        mesh=mesh,
        compiler_params=pltpu.CompilerParams(use_tc_tiling_on_sc=False),
        scratch_types=[
            pltpu.VMEM((nch, CHUNK), jnp.int32),
            pltpu.VMEM((CHUNK, D), jnp.float32),
            pltpu.SemaphoreType.DMA,
        ],
    )
    def emb_kernel(x_hbm, tab_hbm, out_hbm, idx_v, rows, sem):
        wid = lax.axis_index("s") * 2 + lax.axis_index("c")
        base = wid * per_w
        pltpu.sync_copy(x_hbm.at[wid], idx_v)

        def chunk_body(g, carry):
            pltpu.async_copy(tab_hbm.at[idx_v.at[g]], rows, sem).wait()

            def scale_body(i, c):
                for j in range(D // LANES):
                    sl = (i, pl.ds(j * LANES, LANES))
                    rows[sl] = rows[sl] * SCALE
                return c

            lax.fori_loop(0, CHUNK, scale_body, 0, unroll=4)
            pltpu.sync_copy(rows, out_hbm.at[pl.ds(base + g * CHUNK, CHUNK)])
            return carry

        lax.fori_loop(0, nch, chunk_body, 0)

    return emb_kernel


def kernel(x, embedding_weight):
    b, l = x.shape
    n = b * l
    x2 = x.reshape(NW, n // (NW * CHUNK), CHUNK).astype(jnp.int32)
    out = _build(n)(x2, embedding_weight)
    return out.reshape(b, l, D)


# speedup vs baseline: 1.0719x; 1.0719x over previous
import functools
import math

import jax
import jax.numpy as jnp
from jax import lax
from jax.experimental import pallas as pl
from jax.experimental.pallas import tpu as pltpu
from jax.experimental.pallas import tpu_sc as plsc

D = 64
NW = 32
CHUNK = 256
NBUF = 4
LANES = 16
SCALE = math.sqrt(D)


def _build(n_idx):
    per_w = n_idx // NW
    nch = per_w // CHUNK
    assert nch % NBUF == 0
    mesh = plsc.VectorSubcoreMesh(core_axis_name="c", subcore_axis_name="s")

    @functools.partial(
        pl.kernel,
        out_type=jax.ShapeDtypeStruct((n_idx, D), jnp.float32),
        mesh=mesh,
        compiler_params=pltpu.CompilerParams(use_tc_tiling_on_sc=False),
        scratch_types=[
            pltpu.VMEM((nch, CHUNK), jnp.int32),
            *[pltpu.VMEM((CHUNK, D), jnp.float32) for _ in range(NBUF)],
            *[pltpu.SemaphoreType.DMA for _ in range(2 * NBUF)],
        ],
    )
    def emb_kernel(x_hbm, tab_hbm, out_hbm, idx_v, *bufs_sems):
        rows = bufs_sems[:NBUF]
        semg = bufs_sems[NBUF:2 * NBUF]
        semw = bufs_sems[2 * NBUF:]
        wid = lax.axis_index("s") * 2 + lax.axis_index("c")
        base = wid * per_w
        pltpu.sync_copy(x_hbm.at[wid], idx_v)

        def start_gather(g, b):
            pltpu.async_copy(tab_hbm.at[idx_v.at[g]], rows[b], semg[b])

        def wait_gather(g, b):
            pltpu.make_async_copy(tab_hbm.at[idx_v.at[g]], rows[b], semg[b]).wait()

        def start_wb(g, b):
            pltpu.async_copy(rows[b], out_hbm.at[pl.ds(base + g * CHUNK, CHUNK)],
                             semw[b])

        def wait_wb(b):
            pltpu.make_async_copy(
                rows[b], out_hbm.at[pl.ds(base, CHUNK)], semw[b]).wait()

        def scale(b):
            def body(i, c):
                for j in range(D // LANES):
                    sl = (i, pl.ds(j * LANES, LANES))
                    rows[b][sl] = rows[b][sl] * SCALE
                return c
            lax.fori_loop(0, CHUNK, body, 0, unroll=4)

        for b in range(NBUF - 1):
            start_gather(b, b)

        def group_body(t, carry):
            for b in range(NBUF):
                g = NBUF * t + b
                wait_gather(g, b)
                scale(b)
                start_wb(g, b)
                nb = (b - 1) % NBUF
                gn = g + NBUF - 1
                if b == 0:
                    @pl.when(t > 0)
                    def _():
                        wait_wb(nb)
                else:
                    wait_wb(nb)

                @pl.when(gn < nch)
                def _():
                    start_gather(gn, nb)
            return carry

        lax.fori_loop(0, nch // NBUF, group_body, 0)
        wait_wb(NBUF - 1)

    return emb_kernel


def kernel(x, embedding_weight):
    b, l = x.shape
    n = b * l
    x3 = x.reshape(NW, n // (NW * CHUNK), CHUNK).astype(jnp.int32)
    out = _build(n)(x3, embedding_weight)
    return out.reshape(b, l, D)
